# Initial kernel scaffold; baseline (speedup 1.0000x reference)
import jax
import jax.numpy as jnp
from jax.experimental import pallas as pl


def kernel(weights, x, points, edge_index):
    raise NotImplementedError("write your pallas kernel here")


# speedup vs baseline: 9.6587x; 9.6587x over previous
import functools
import math

import jax
import jax.numpy as jnp
import numpy as np
from jax import lax
from jax.experimental import pallas as pl
from jax.experimental.pallas import tpu as pltpu
from jax.experimental.pallas import tpu_sc as plsc

N = 10000
E = 160000
CI = 16
CO = 16
SI = 8
SO = 8
R = 8
A = 6
K = R * A
D_IN = CI * SI
D_OUT = CO * SO

_rng = np.random.default_rng(42)
_T_NP = _rng.standard_normal((A, SO, SI)).astype(np.float32) / np.sqrt(SI)
_MU = np.linspace(0.0, 2.0, R, dtype=np.float32)
_SIGMA = 0.5

_NC = 2
_NS = 16
_NW = _NC * _NS
_PER_W = E // _NW
_CHUNK = 128
_NFULL = _PER_W // _CHUNK
_TAIL = _PER_W - _NFULL * _CHUNK


def _gather_body(x_hbm, src_hbm, out_hbm, idx_v, rows_v, rows_t, sem):
    wid = lax.axis_index("s") * _NC + lax.axis_index("c")
    base = wid * _PER_W
    pltpu.sync_copy(src_hbm.at[pl.ds(base, _PER_W)], idx_v)

    def body(i, _):
        off = i * _CHUNK
        idx = idx_v.at[pl.ds(off, _CHUNK)]
        pltpu.async_copy(x_hbm.at[idx], rows_v, sem).wait()
        pltpu.sync_copy(rows_v, out_hbm.at[pl.ds(base + off, _CHUNK)])
        return 0

    lax.fori_loop(0, _NFULL, body, 0)
    toff = _NFULL * _CHUNK
    tidx = idx_v.at[pl.ds(toff, _TAIL)]
    pltpu.async_copy(x_hbm.at[tidx], rows_t, sem).wait()
    pltpu.sync_copy(rows_t, out_hbm.at[pl.ds(base + toff, _TAIL)])


def _gather_call(x, src):
    f = functools.partial(
        pl.kernel,
        out_type=jax.ShapeDtypeStruct((E, D_IN), jnp.float32),
        mesh=plsc.VectorSubcoreMesh(core_axis_name="c", subcore_axis_name="s",
                                    num_cores=_NC, num_subcores=_NS),
        scratch_types=[
            pltpu.VMEM((_PER_W,), jnp.int32),
            pltpu.VMEM((_CHUNK, D_IN), jnp.float32),
            pltpu.VMEM((_TAIL, D_IN), jnp.float32),
            pltpu.SemaphoreType.DMA,
        ],
    )(_gather_body)
    return f(x, src)


_P = 256


def _compute_body(xj_ref, pts_ref, w_ref, out_ref):
    xj = xj_ref[...]
    pts = pts_ref[...]
    px = pts[:, 0:1]
    py = pts[:, 1:2]
    pz = pts[:, 2:3]
    r = jnp.sqrt(px * px + py * py + pz * pz)
    inv = 1.0 / (r + 1e-8)
    nx = px * inv
    ny = py * inv
    nz = pz * inv
    ms = [
        jnp.ones_like(nx),
        nx,
        ny,
        nz,
        nx * ny,
        nz * nz - jnp.float32(1.0 / 3.0),
    ]
    gs = [jnp.exp(-((r - jnp.float32(mu)) ** 2) * jnp.float32(1.0 / (2.0 * _SIGMA**2)))
          for mu in _MU]

    xjb = xj.astype(jnp.bfloat16)
    acc = jnp.zeros((_P, D_OUT), dtype=jnp.float32)
    for k in range(K):
        rr, aa = divmod(k, A)
        gm = (gs[rr] * ms[aa]).astype(jnp.bfloat16)
        t = xjb * gm
        acc = acc + jnp.dot(t, w_ref[k * D_IN:(k + 1) * D_IN, :],
                            preferred_element_type=jnp.float32)
    out_ref[...] = acc


def _compute_msg(xj, points, wbig):
    return pl.pallas_call(
        _compute_body,
        grid=(E // _P,),
        in_specs=[
            pl.BlockSpec((_P, D_IN), lambda i: (i, 0)),
            pl.BlockSpec((_P, 3), lambda i: (i, 0)),
            pl.BlockSpec((K * D_IN, D_OUT), lambda i: (0, 0)),
        ],
        out_specs=pl.BlockSpec((_P, D_OUT), lambda i: (i, 0)),
        out_shape=jax.ShapeDtypeStruct((E, D_OUT), jnp.float32),
    )(xj, points, wbig)


_ROWS_PER_SUB = 624
_ROWS_TAIL_OFF = _ROWS_PER_SUB * _NS
_ROWS_TAIL = N - _ROWS_TAIL_OFF


def _scatter_body(msg_hbm, dst_hbm, zeros_hbm, out_hbm,
                  idx2_v, idxt_v, rows_v, rows_t, part_sh):
    cid = lax.axis_index("c")
    sid = lax.axis_index("s")
    wid = sid * _NC + cid
    pltpu.sync_copy(zeros_hbm.at[pl.ds(sid * _ROWS_PER_SUB, _ROWS_PER_SUB)],
                    part_sh.at[pl.ds(sid * _ROWS_PER_SUB, _ROWS_PER_SUB)])

    @pl.when(sid == _NS - 1)
    def _zero_tail():
        pltpu.sync_copy(zeros_hbm.at[pl.ds(_ROWS_TAIL_OFF, _ROWS_TAIL)],
                        part_sh.at[pl.ds(_ROWS_TAIL_OFF, _ROWS_TAIL)])

    plsc.subcore_barrier()

    base = wid * _PER_W

    def body(i, _):
        off = base + i * _CHUNK
        pltpu.sync_copy(dst_hbm.at[pl.ds(off, _CHUNK)], idx2_v.at[0])
        pltpu.sync_copy(msg_hbm.at[pl.ds(off, _CHUNK)], rows_v)
        pltpu.sync_copy(rows_v, part_sh.at[idx2_v.at[0]], add=True)
        return 0

    lax.fori_loop(0, _NFULL, body, 0)
    toff = base + _NFULL * _CHUNK
    pltpu.sync_copy(dst_hbm.at[pl.ds(toff, _TAIL)], idxt_v)
    pltpu.sync_copy(msg_hbm.at[pl.ds(toff, _TAIL)], rows_t)
    pltpu.sync_copy(rows_t, part_sh.at[idxt_v], add=True)

    plsc.subcore_barrier()
    row = sid * _ROWS_PER_SUB
    pltpu.sync_copy(part_sh.at[pl.ds(row, _ROWS_PER_SUB)],
                    out_hbm.at[pl.ds(cid * N + row, _ROWS_PER_SUB)])

    @pl.when(sid == _NS - 1)
    def _write_tail():
        pltpu.sync_copy(part_sh.at[pl.ds(_ROWS_TAIL_OFF, _ROWS_TAIL)],
                        out_hbm.at[pl.ds(cid * N + _ROWS_TAIL_OFF, _ROWS_TAIL)])


def _scatter_call(msg, dst, zeros):
    f = functools.partial(
        pl.kernel,
        out_type=jax.ShapeDtypeStruct((2 * N, D_OUT), jnp.float32),
        mesh=plsc.VectorSubcoreMesh(core_axis_name="c", subcore_axis_name="s",
                                    num_cores=_NC, num_subcores=_NS),
        scratch_types=[
            pltpu.VMEM((1, _CHUNK), jnp.int32),
            pltpu.VMEM((_TAIL,), jnp.int32),
            pltpu.VMEM((_CHUNK, D_OUT), jnp.float32),
            pltpu.VMEM((_TAIL, D_OUT), jnp.float32),
            pltpu.VMEM_SHARED((N, D_OUT), jnp.float32),
        ],
    )(_scatter_body)
    return f(msg, dst, zeros)


_MB = 1000


def _merge_body(a_ref, b_ref, out_ref):
    out_ref[...] = a_ref[...] + b_ref[...]


def _merge(partials):
    return pl.pallas_call(
        _merge_body,
        grid=(N // _MB,),
        in_specs=[
            pl.BlockSpec((_MB, D_OUT), lambda i: (i, 0)),
            pl.BlockSpec((_MB, D_OUT), lambda i: (i + N // _MB, 0)),
        ],
        out_specs=pl.BlockSpec((_MB, D_OUT), lambda i: (i, 0)),
        out_shape=jax.ShapeDtypeStruct((N, D_OUT), jnp.float32),
    )(partials, partials)


def _build_wbig(weights):
    c = weights.reshape(CO, CI, R, A)
    c_t = jnp.transpose(c, (2, 3, 1, 0))
    t_t = jnp.transpose(jnp.asarray(_T_NP), (0, 2, 1))
    w = (c_t[:, :, :, None, :, None] * t_t[None, :, None, :, None, :])
    return w.reshape(K * D_IN, CO * SO).astype(jnp.bfloat16)


def kernel(weights, x, points, edge_index):
    src = edge_index[0].astype(jnp.int32)
    dst = edge_index[1].astype(jnp.int32)
    wbig = _build_wbig(weights)
    zeros = jnp.zeros((N, D_OUT), dtype=jnp.float32)

    xj = _gather_call(x, src)
    msg = _compute_msg(xj, points, wbig)
    partials = _scatter_call(msg, dst, zeros)
    return _merge(partials)


# speedup vs baseline: 11.3414x; 1.1742x over previous
import functools
import math

import jax
import jax.numpy as jnp
import numpy as np
from jax import lax
from jax.experimental import pallas as pl
from jax.experimental.pallas import tpu as pltpu
from jax.experimental.pallas import tpu_sc as plsc

N = 10000
E = 160000
CI = 16
CO = 16
SI = 8
SO = 8
R = 8
A = 6
K = R * A
D_IN = CI * SI
D_OUT = CO * SO

_rng = np.random.default_rng(42)
_T_NP = _rng.standard_normal((A, SO, SI)).astype(np.float32) / np.sqrt(SI)
_MU = np.linspace(0.0, 2.0, R, dtype=np.float32)
_SIGMA = 0.5

_NC = 2
_NS = 16
_NW = _NC * _NS
_PER_W = E // _NW
_CHUNK = 128
_NFULL = _PER_W // _CHUNK
_TAIL = _PER_W - _NFULL * _CHUNK


def _gather_body(x_hbm, src_hbm, out_hbm, idx_v, rows_v, rows_t, sem):
    wid = lax.axis_index("s") * _NC + lax.axis_index("c")
    base = wid * _PER_W
    pltpu.sync_copy(src_hbm.at[pl.ds(base, _PER_W)], idx_v)

    def body(i, _):
        off = i * _CHUNK
        idx = idx_v.at[pl.ds(off, _CHUNK)]
        pltpu.async_copy(x_hbm.at[idx], rows_v, sem).wait()
        pltpu.sync_copy(rows_v, out_hbm.at[pl.ds(base + off, _CHUNK)])
        return 0

    lax.fori_loop(0, _NFULL, body, 0)
    toff = _NFULL * _CHUNK
    tidx = idx_v.at[pl.ds(toff, _TAIL)]
    pltpu.async_copy(x_hbm.at[tidx], rows_t, sem).wait()
    pltpu.sync_copy(rows_t, out_hbm.at[pl.ds(base + toff, _TAIL)])


def _gather_call(x, src):
    f = functools.partial(
        pl.kernel,
        out_type=jax.ShapeDtypeStruct((E, D_IN), jnp.float32),
        mesh=plsc.VectorSubcoreMesh(core_axis_name="c", subcore_axis_name="s",
                                    num_cores=_NC, num_subcores=_NS),
        scratch_types=[
            pltpu.VMEM((_PER_W,), jnp.int32),
            pltpu.VMEM((_CHUNK, D_IN), jnp.float32),
            pltpu.VMEM((_TAIL, D_IN), jnp.float32),
            pltpu.SemaphoreType.DMA,
        ],
    )(_gather_body)
    return f(x, src)


_P = 512


def _compute_body(xj_ref, pts_ref, w_ref, out_ref, t2_ref):
    xj = xj_ref[...]
    pts = pts_ref[...]
    px = pts[:, 0:1]
    py = pts[:, 1:2]
    pz = pts[:, 2:3]
    r = jnp.sqrt(px * px + py * py + pz * pz)
    inv = 1.0 / (r + 1e-8)
    nx = px * inv
    ny = py * inv
    nz = pz * inv
    ms = [
        None,
        nx.astype(jnp.bfloat16),
        ny.astype(jnp.bfloat16),
        nz.astype(jnp.bfloat16),
        (nx * ny).astype(jnp.bfloat16),
        (nz * nz - jnp.float32(1.0 / 3.0)).astype(jnp.bfloat16),
    ]
    gs = [jnp.exp(-((r - jnp.float32(mu)) ** 2) * jnp.float32(1.0 / (2.0 * _SIGMA**2)))
          for mu in _MU]

    xjb = xj.astype(jnp.bfloat16)
    xg = [xjb * g.astype(jnp.bfloat16) for g in gs]
    for k in range(K):
        rr, aa = divmod(k, A)
        t = xg[rr] if aa == 0 else xg[rr] * ms[aa]
        t2_ref[:, k * D_IN:(k + 1) * D_IN] = t
    out_ref[...] = jnp.dot(t2_ref[...], w_ref[...],
                           preferred_element_type=jnp.float32)


def _compute_msg(xj, points, wbig):
    return pl.pallas_call(
        _compute_body,
        grid=(E // _P,),
        in_specs=[
            pl.BlockSpec((_P, D_IN), lambda i: (i, 0)),
            pl.BlockSpec((_P, 3), lambda i: (i, 0)),
            pl.BlockSpec((K * D_IN, D_OUT), lambda i: (0, 0)),
        ],
        out_specs=pl.BlockSpec((_P, D_OUT), lambda i: (i, 0)),
        out_shape=jax.ShapeDtypeStruct((E, D_OUT), jnp.float32),
        scratch_shapes=[pltpu.VMEM((_P, K * D_IN), jnp.bfloat16)],
    )(xj, points, wbig)


_ROWS_PER_SUB = 624
_ROWS_TAIL_OFF = _ROWS_PER_SUB * _NS
_ROWS_TAIL = N - _ROWS_TAIL_OFF


def _scatter_body(msg_hbm, dst_hbm, zeros_hbm, out_hbm,
                  idx2_v, idxt_v, rows_v, rows_t, part_sh):
    cid = lax.axis_index("c")
    sid = lax.axis_index("s")
    wid = sid * _NC + cid
    pltpu.sync_copy(zeros_hbm.at[pl.ds(sid * _ROWS_PER_SUB, _ROWS_PER_SUB)],
                    part_sh.at[pl.ds(sid * _ROWS_PER_SUB, _ROWS_PER_SUB)])

    @pl.when(sid == _NS - 1)
    def _zero_tail():
        pltpu.sync_copy(zeros_hbm.at[pl.ds(_ROWS_TAIL_OFF, _ROWS_TAIL)],
                        part_sh.at[pl.ds(_ROWS_TAIL_OFF, _ROWS_TAIL)])

    plsc.subcore_barrier()

    base = wid * _PER_W

    def body(i, _):
        off = base + i * _CHUNK
        pltpu.sync_copy(dst_hbm.at[pl.ds(off, _CHUNK)], idx2_v.at[0])
        pltpu.sync_copy(msg_hbm.at[pl.ds(off, _CHUNK)], rows_v)
        pltpu.sync_copy(rows_v, part_sh.at[idx2_v.at[0]], add=True)
        return 0

    lax.fori_loop(0, _NFULL, body, 0)
    toff = base + _NFULL * _CHUNK
    pltpu.sync_copy(dst_hbm.at[pl.ds(toff, _TAIL)], idxt_v)
    pltpu.sync_copy(msg_hbm.at[pl.ds(toff, _TAIL)], rows_t)
    pltpu.sync_copy(rows_t, part_sh.at[idxt_v], add=True)

    plsc.subcore_barrier()
    row = sid * _ROWS_PER_SUB
    pltpu.sync_copy(part_sh.at[pl.ds(row, _ROWS_PER_SUB)],
                    out_hbm.at[pl.ds(cid * N + row, _ROWS_PER_SUB)])

    @pl.when(sid == _NS - 1)
    def _write_tail():
        pltpu.sync_copy(part_sh.at[pl.ds(_ROWS_TAIL_OFF, _ROWS_TAIL)],
                        out_hbm.at[pl.ds(cid * N + _ROWS_TAIL_OFF, _ROWS_TAIL)])


def _scatter_call(msg, dst, zeros):
    f = functools.partial(
        pl.kernel,
        out_type=jax.ShapeDtypeStruct((2 * N, D_OUT), jnp.float32),
        mesh=plsc.VectorSubcoreMesh(core_axis_name="c", subcore_axis_name="s",
                                    num_cores=_NC, num_subcores=_NS),
        scratch_types=[
            pltpu.VMEM((1, _CHUNK), jnp.int32),
            pltpu.VMEM((_TAIL,), jnp.int32),
            pltpu.VMEM((_CHUNK, D_OUT), jnp.float32),
            pltpu.VMEM((_TAIL, D_OUT), jnp.float32),
            pltpu.VMEM_SHARED((N, D_OUT), jnp.float32),
        ],
    )(_scatter_body)
    return f(msg, dst, zeros)


_MB = 1000


def _merge_body(a_ref, b_ref, out_ref):
    out_ref[...] = a_ref[...] + b_ref[...]


def _merge(partials):
    return pl.pallas_call(
        _merge_body,
        grid=(N // _MB,),
        in_specs=[
            pl.BlockSpec((_MB, D_OUT), lambda i: (i, 0)),
            pl.BlockSpec((_MB, D_OUT), lambda i: (i + N // _MB, 0)),
        ],
        out_specs=pl.BlockSpec((_MB, D_OUT), lambda i: (i, 0)),
        out_shape=jax.ShapeDtypeStruct((N, D_OUT), jnp.float32),
    )(partials, partials)


def _build_wbig(weights):
    c = weights.reshape(CO, CI, R, A)
    c_t = jnp.transpose(c, (2, 3, 1, 0))
    t_t = jnp.transpose(jnp.asarray(_T_NP), (0, 2, 1))
    w = (c_t[:, :, :, None, :, None] * t_t[None, :, None, :, None, :])
    return w.reshape(K * D_IN, CO * SO).astype(jnp.bfloat16)


def kernel(weights, x, points, edge_index):
    src = edge_index[0].astype(jnp.int32)
    dst = edge_index[1].astype(jnp.int32)
    wbig = _build_wbig(weights)
    zeros = jnp.zeros((N, D_OUT), dtype=jnp.float32)

    xj = _gather_call(x, src)
    msg = _compute_msg(xj, points, wbig)
    partials = _scatter_call(msg, dst, zeros)
    return _merge(partials)


# speedup vs baseline: 13.6486x; 1.2034x over previous
import functools
import math

import jax
import jax.numpy as jnp
import numpy as np
from jax import lax
from jax.experimental import pallas as pl
from jax.experimental.pallas import tpu as pltpu
from jax.experimental.pallas import tpu_sc as plsc

N = 10000
E = 160000
CI = 16
CO = 16
SI = 8
SO = 8
R = 8
A = 6
K = R * A
D_IN = CI * SI
D_OUT = CO * SO

_rng = np.random.default_rng(42)
_T_NP = _rng.standard_normal((A, SO, SI)).astype(np.float32) / np.sqrt(SI)
_MU = np.linspace(0.0, 2.0, R, dtype=np.float32)
_SIGMA = 0.5

_NC = 2
_NS = 16
_NW = _NC * _NS
_PER_W = E // _NW
_CHUNK = 128
_NFULL = _PER_W // _CHUNK
_TAIL = _PER_W - _NFULL * _CHUNK


def _gather_body(x_hbm, src_hbm, out_hbm, idx_v, rows_v, rows_t, sem):
    wid = lax.axis_index("s") * _NC + lax.axis_index("c")
    base = wid * _PER_W
    pltpu.sync_copy(src_hbm.at[pl.ds(base, _PER_W)], idx_v)

    def body(i, _):
        off = i * _CHUNK
        idx = idx_v.at[pl.ds(off, _CHUNK)]
        pltpu.async_copy(x_hbm.at[idx], rows_v, sem).wait()
        pltpu.sync_copy(rows_v, out_hbm.at[pl.ds(base + off, _CHUNK)])
        return 0

    lax.fori_loop(0, _NFULL, body, 0)
    toff = _NFULL * _CHUNK
    tidx = idx_v.at[pl.ds(toff, _TAIL)]
    pltpu.async_copy(x_hbm.at[tidx], rows_t, sem).wait()
    pltpu.sync_copy(rows_t, out_hbm.at[pl.ds(base + toff, _TAIL)])


def _gather_call(x, src):
    f = functools.partial(
        pl.kernel,
        out_type=jax.ShapeDtypeStruct((E, D_IN), jnp.float32),
        mesh=plsc.VectorSubcoreMesh(core_axis_name="c", subcore_axis_name="s",
                                    num_cores=_NC, num_subcores=_NS),
        scratch_types=[
            pltpu.VMEM((_PER_W,), jnp.int32),
            pltpu.VMEM((_CHUNK, D_IN), jnp.float32),
            pltpu.VMEM((_TAIL, D_IN), jnp.float32),
            pltpu.SemaphoreType.DMA,
        ],
    )(_gather_body)
    return f(x, src)


_P = 640


def _compute_body(xj_ref, pts_ref, w_ref, out_ref):
    xj = xj_ref[...]
    pts = pts_ref[...]
    px = pts[:, 0:1]
    py = pts[:, 1:2]
    pz = pts[:, 2:3]
    r = jnp.sqrt(px * px + py * py + pz * pz)
    inv = 1.0 / (r + 1e-8)
    nx = px * inv
    ny = py * inv
    nz = pz * inv
    ms = [
        None,
        nx.astype(jnp.bfloat16),
        ny.astype(jnp.bfloat16),
        nz.astype(jnp.bfloat16),
        (nx * ny).astype(jnp.bfloat16),
        (nz * nz - jnp.float32(1.0 / 3.0)).astype(jnp.bfloat16),
    ]
    gs = [jnp.exp(-((r - jnp.float32(mu)) ** 2) * jnp.float32(1.0 / (2.0 * _SIGMA**2)))
          for mu in _MU]

    xjb = xj.astype(jnp.bfloat16)
    xg = [xjb * g.astype(jnp.bfloat16) for g in gs]
    pieces = []
    for k in range(K):
        rr, aa = divmod(k, A)
        pieces.append(xg[rr] if aa == 0 else xg[rr] * ms[aa])
    t2 = jnp.concatenate(pieces, axis=1)
    out_ref[...] = jnp.dot(t2, w_ref[...],
                           preferred_element_type=jnp.float32)


def _compute_msg(xj, points, wbig):
    return pl.pallas_call(
        _compute_body,
        grid=(E // _P,),
        in_specs=[
            pl.BlockSpec((_P, D_IN), lambda i: (i, 0)),
            pl.BlockSpec((_P, 3), lambda i: (i, 0)),
            pl.BlockSpec((K * D_IN, D_OUT), lambda i: (0, 0)),
        ],
        out_specs=pl.BlockSpec((_P, D_OUT), lambda i: (i, 0)),
        out_shape=jax.ShapeDtypeStruct((E, D_OUT), jnp.float32),
    )(xj, points, wbig)


_ROWS_PER_SUB = 624
_ROWS_TAIL_OFF = _ROWS_PER_SUB * _NS
_ROWS_TAIL = N - _ROWS_TAIL_OFF


def _scatter_body(msg_hbm, dst_hbm, zeros_hbm, out_hbm,
                  idx2_v, idxt_v, rows_v, rows_t, part_sh):
    cid = lax.axis_index("c")
    sid = lax.axis_index("s")
    wid = sid * _NC + cid
    pltpu.sync_copy(zeros_hbm.at[pl.ds(sid * _ROWS_PER_SUB, _ROWS_PER_SUB)],
                    part_sh.at[pl.ds(sid * _ROWS_PER_SUB, _ROWS_PER_SUB)])

    @pl.when(sid == _NS - 1)
    def _zero_tail():
        pltpu.sync_copy(zeros_hbm.at[pl.ds(_ROWS_TAIL_OFF, _ROWS_TAIL)],
                        part_sh.at[pl.ds(_ROWS_TAIL_OFF, _ROWS_TAIL)])

    plsc.subcore_barrier()

    base = wid * _PER_W

    def body(i, _):
        off = base + i * _CHUNK
        pltpu.sync_copy(dst_hbm.at[pl.ds(off, _CHUNK)], idx2_v.at[0])
        pltpu.sync_copy(msg_hbm.at[pl.ds(off, _CHUNK)], rows_v)
        pltpu.sync_copy(rows_v, part_sh.at[idx2_v.at[0]], add=True)
        return 0

    lax.fori_loop(0, _NFULL, body, 0)
    toff = base + _NFULL * _CHUNK
    pltpu.sync_copy(dst_hbm.at[pl.ds(toff, _TAIL)], idxt_v)
    pltpu.sync_copy(msg_hbm.at[pl.ds(toff, _TAIL)], rows_t)
    pltpu.sync_copy(rows_t, part_sh.at[idxt_v], add=True)

    plsc.subcore_barrier()
    row = sid * _ROWS_PER_SUB
    pltpu.sync_copy(part_sh.at[pl.ds(row, _ROWS_PER_SUB)],
                    out_hbm.at[pl.ds(cid * N + row, _ROWS_PER_SUB)])

    @pl.when(sid == _NS - 1)
    def _write_tail():
        pltpu.sync_copy(part_sh.at[pl.ds(_ROWS_TAIL_OFF, _ROWS_TAIL)],
                        out_hbm.at[pl.ds(cid * N + _ROWS_TAIL_OFF, _ROWS_TAIL)])


def _scatter_call(msg, dst, zeros):
    f = functools.partial(
        pl.kernel,
        out_type=jax.ShapeDtypeStruct((2 * N, D_OUT), jnp.float32),
        mesh=plsc.VectorSubcoreMesh(core_axis_name="c", subcore_axis_name="s",
                                    num_cores=_NC, num_subcores=_NS),
        scratch_types=[
            pltpu.VMEM((1, _CHUNK), jnp.int32),
            pltpu.VMEM((_TAIL,), jnp.int32),
            pltpu.VMEM((_CHUNK, D_OUT), jnp.float32),
            pltpu.VMEM((_TAIL, D_OUT), jnp.float32),
            pltpu.VMEM_SHARED((N, D_OUT), jnp.float32),
        ],
    )(_scatter_body)
    return f(msg, dst, zeros)


_MB = 1000


def _merge_body(a_ref, b_ref, out_ref):
    out_ref[...] = a_ref[...] + b_ref[...]


def _merge(partials):
    return pl.pallas_call(
        _merge_body,
        grid=(N // _MB,),
        in_specs=[
            pl.BlockSpec((_MB, D_OUT), lambda i: (i, 0)),
            pl.BlockSpec((_MB, D_OUT), lambda i: (i + N // _MB, 0)),
        ],
        out_specs=pl.BlockSpec((_MB, D_OUT), lambda i: (i, 0)),
        out_shape=jax.ShapeDtypeStruct((N, D_OUT), jnp.float32),
    )(partials, partials)


def _build_wbig(weights):
    c = weights.reshape(CO, CI, R, A)
    c_t = jnp.transpose(c, (2, 3, 1, 0))
    t_t = jnp.transpose(jnp.asarray(_T_NP), (0, 2, 1))
    w = (c_t[:, :, :, None, :, None] * t_t[None, :, None, :, None, :])
    return w.reshape(K * D_IN, CO * SO).astype(jnp.bfloat16)


def kernel(weights, x, points, edge_index):
    src = edge_index[0].astype(jnp.int32)
    dst = edge_index[1].astype(jnp.int32)
    wbig = _build_wbig(weights)
    zeros = jnp.zeros((N, D_OUT), dtype=jnp.float32)

    xj = _gather_call(x, src)
    msg = _compute_msg(xj, points, wbig)
    partials = _scatter_call(msg, dst, zeros)
    return _merge(partials)


# speedup vs baseline: 14.3515x; 1.0515x over previous
import functools

import jax
import jax.numpy as jnp
import numpy as np
from jax import lax
from jax.experimental import pallas as pl
from jax.experimental.pallas import tpu as pltpu
from jax.experimental.pallas import tpu_sc as plsc

N = 10000
E = 160000
CI = 16
CO = 16
SI = 8
SO = 8
R = 8
A = 6
K = R * A
D_IN = CI * SI
D_OUT = CO * SO

_rng = np.random.default_rng(42)
_T_NP = _rng.standard_normal((A, SO, SI)).astype(np.float32) / np.sqrt(SI)
_MU = np.linspace(0.0, 2.0, R, dtype=np.float32)
_SIGMA = 0.5

_NC = 2
_NS = 16
_NW = _NC * _NS
_CHUNK = 128

_E1 = 81920
_E2 = E - _E1


def _mesh():
    return plsc.VectorSubcoreMesh(core_axis_name="c", subcore_axis_name="s",
                                  num_cores=_NC, num_subcores=_NS)


def _gather_body(per_w, nfull, tail,
                 x_hbm, src_hbm, out_hbm, idx_v, rows_v, rows_t, sem):
    wid = lax.axis_index("s") * _NC + lax.axis_index("c")
    base = wid * per_w
    pltpu.sync_copy(src_hbm.at[pl.ds(base, per_w)], idx_v)

    def body(i, _):
        off = i * _CHUNK
        idx = idx_v.at[pl.ds(off, _CHUNK)]
        pltpu.async_copy(x_hbm.at[idx], rows_v, sem).wait()
        pltpu.sync_copy(rows_v, out_hbm.at[pl.ds(base + off, _CHUNK)])
        return 0

    lax.fori_loop(0, nfull, body, 0)
    if tail:
        toff = nfull * _CHUNK
        tidx = idx_v.at[pl.ds(toff, tail)]
        pltpu.async_copy(x_hbm.at[tidx], rows_t, sem).wait()
        pltpu.sync_copy(rows_t, out_hbm.at[pl.ds(base + toff, tail)])


def _gather_call(x, src, e_sub):
    per_w = e_sub // _NW
    nfull, tail = divmod(per_w, _CHUNK)
    f = functools.partial(
        pl.kernel,
        out_type=jax.ShapeDtypeStruct((e_sub, D_IN), jnp.float32),
        mesh=_mesh(),
        scratch_types=[
            pltpu.VMEM((per_w,), jnp.int32),
            pltpu.VMEM((_CHUNK, D_IN), jnp.float32),
            pltpu.VMEM((max(tail, 8), D_IN), jnp.float32),
            pltpu.SemaphoreType.DMA,
        ],
    )(functools.partial(_gather_body, per_w, nfull, tail))
    return f(x, src)


_P = 640


def _compute_body(xj_ref, pts_ref, w_ref, out_ref):
    xj = xj_ref[...]
    pts = pts_ref[...]
    px = pts[:, 0:1]
    py = pts[:, 1:2]
    pz = pts[:, 2:3]
    r = jnp.sqrt(px * px + py * py + pz * pz)
    inv = 1.0 / (r + 1e-8)
    nx = px * inv
    ny = py * inv
    nz = pz * inv
    ms = [
        None,
        nx.astype(jnp.bfloat16),
        ny.astype(jnp.bfloat16),
        nz.astype(jnp.bfloat16),
        (nx * ny).astype(jnp.bfloat16),
        (nz * nz - jnp.float32(1.0 / 3.0)).astype(jnp.bfloat16),
    ]
    gs = [jnp.exp(-((r - jnp.float32(mu)) ** 2) * jnp.float32(1.0 / (2.0 * _SIGMA**2)))
          for mu in _MU]

    xjb = xj.astype(jnp.bfloat16)
    xg = [xjb * g.astype(jnp.bfloat16) for g in gs]
    pieces = []
    for k in range(K):
        rr, aa = divmod(k, A)
        pieces.append(xg[rr] if aa == 0 else xg[rr] * ms[aa])
    t2 = jnp.concatenate(pieces, axis=1)
    out_ref[...] = jnp.dot(t2, w_ref[...],
                           preferred_element_type=jnp.float32)


def _compute_msg(xj, points, wbig):
    e_sub = xj.shape[0]
    return pl.pallas_call(
        _compute_body,
        grid=(e_sub // _P,),
        in_specs=[
            pl.BlockSpec((_P, D_IN), lambda i: (i, 0)),
            pl.BlockSpec((_P, 3), lambda i: (i, 0)),
            pl.BlockSpec((K * D_IN, D_OUT), lambda i: (0, 0)),
        ],
        out_specs=pl.BlockSpec((_P, D_OUT), lambda i: (i, 0)),
        out_shape=jax.ShapeDtypeStruct((e_sub, D_OUT), jnp.float32),
    )(xj, points, wbig)


_ROWS_PER_SUB = 624
_ROWS_TAIL_OFF = _ROWS_PER_SUB * _NS
_ROWS_TAIL = N - _ROWS_TAIL_OFF


def _scatter_body(per_w, nfull, tail,
                  msg_hbm, dst_hbm, zeros_hbm, out_hbm,
                  idx2_v, idxt_v, rows_v, rows_t, part_sh):
    cid = lax.axis_index("c")
    sid = lax.axis_index("s")
    wid = sid * _NC + cid
    pltpu.sync_copy(zeros_hbm.at[pl.ds(sid * _ROWS_PER_SUB, _ROWS_PER_SUB)],
                    part_sh.at[pl.ds(sid * _ROWS_PER_SUB, _ROWS_PER_SUB)])

    @pl.when(sid == _NS - 1)
    def _zero_tail():
        pltpu.sync_copy(zeros_hbm.at[pl.ds(_ROWS_TAIL_OFF, _ROWS_TAIL)],
                        part_sh.at[pl.ds(_ROWS_TAIL_OFF, _ROWS_TAIL)])

    plsc.subcore_barrier()

    base = wid * per_w

    def body(i, _):
        off = base + i * _CHUNK
        pltpu.sync_copy(dst_hbm.at[pl.ds(off, _CHUNK)], idx2_v.at[0])
        pltpu.sync_copy(msg_hbm.at[pl.ds(off, _CHUNK)], rows_v)
        pltpu.sync_copy(rows_v, part_sh.at[idx2_v.at[0]], add=True)
        return 0

    lax.fori_loop(0, nfull, body, 0)
    if tail:
        toff = base + nfull * _CHUNK
        pltpu.sync_copy(dst_hbm.at[pl.ds(toff, tail)], idxt_v)
        pltpu.sync_copy(msg_hbm.at[pl.ds(toff, tail)], rows_t)
        pltpu.sync_copy(rows_t, part_sh.at[idxt_v], add=True)

    plsc.subcore_barrier()
    row = sid * _ROWS_PER_SUB
    pltpu.sync_copy(part_sh.at[pl.ds(row, _ROWS_PER_SUB)],
                    out_hbm.at[pl.ds(cid * N + row, _ROWS_PER_SUB)])

    @pl.when(sid == _NS - 1)
    def _write_tail():
        pltpu.sync_copy(part_sh.at[pl.ds(_ROWS_TAIL_OFF, _ROWS_TAIL)],
                        out_hbm.at[pl.ds(cid * N + _ROWS_TAIL_OFF, _ROWS_TAIL)])


def _scatter_call(msg, dst, zeros):
    e_sub = msg.shape[0]
    per_w = e_sub // _NW
    nfull, tail = divmod(per_w, _CHUNK)
    f = functools.partial(
        pl.kernel,
        out_type=jax.ShapeDtypeStruct((2 * N, D_OUT), jnp.float32),
        mesh=_mesh(),
        scratch_types=[
            pltpu.VMEM((1, _CHUNK), jnp.int32),
            pltpu.VMEM((max(tail, 8),), jnp.int32),
            pltpu.VMEM((_CHUNK, D_OUT), jnp.float32),
            pltpu.VMEM((max(tail, 8), D_OUT), jnp.float32),
            pltpu.VMEM_SHARED((N, D_OUT), jnp.float32),
        ],
    )(functools.partial(_scatter_body, per_w, nfull, tail))
    return f(msg, dst, zeros)


_MB = 1000


def _merge_body(a0_ref, a1_ref, b0_ref, b1_ref, out_ref):
    out_ref[...] = (a0_ref[...] + a1_ref[...]) + (b0_ref[...] + b1_ref[...])


def _merge(pa, pb):
    nb = N // _MB
    return pl.pallas_call(
        _merge_body,
        grid=(nb,),
        in_specs=[
            pl.BlockSpec((_MB, D_OUT), lambda i: (i, 0)),
            pl.BlockSpec((_MB, D_OUT), lambda i: (i + nb, 0)),
            pl.BlockSpec((_MB, D_OUT), lambda i: (i, 0)),
            pl.BlockSpec((_MB, D_OUT), lambda i: (i + nb, 0)),
        ],
        out_specs=pl.BlockSpec((_MB, D_OUT), lambda i: (i, 0)),
        out_shape=jax.ShapeDtypeStruct((N, D_OUT), jnp.float32),
    )(pa, pa, pb, pb)


def _build_wbig(weights):
    c = weights.reshape(CO, CI, R, A)
    c_t = jnp.transpose(c, (2, 3, 1, 0))
    t_t = jnp.transpose(jnp.asarray(_T_NP), (0, 2, 1))
    w = (c_t[:, :, :, None, :, None] * t_t[None, :, None, :, None, :])
    return w.reshape(K * D_IN, CO * SO).astype(jnp.bfloat16)


def kernel(weights, x, points, edge_index):
    src = edge_index[0].astype(jnp.int32)
    dst = edge_index[1].astype(jnp.int32)
    wbig = _build_wbig(weights)
    zeros = jnp.zeros((N, D_OUT), dtype=jnp.float32)

    xj1 = _gather_call(x, src[:_E1], _E1)
    xj2 = _gather_call(x, src[_E1:], _E2)
    msg1 = _compute_msg(xj1, points[:_E1], wbig)
    msg2 = _compute_msg(xj2, points[_E1:], wbig)
    pa = _scatter_call(msg1, dst[:_E1], zeros)
    pb = _scatter_call(msg2, dst[_E1:], zeros)
    return _merge(pa, pb)


# speedup vs baseline: 19.5481x; 1.3621x over previous
import functools

import jax
import jax.numpy as jnp
import numpy as np
from jax import lax
from jax.experimental import pallas as pl
from jax.experimental.pallas import tpu as pltpu
from jax.experimental.pallas import tpu_sc as plsc

N = 10000
E = 160000
CI = 16
CO = 16
SI = 8
SO = 8
R = 8
A = 6
K = R * A
D_IN = CI * SI
D_OUT = CO * SO

_rng = np.random.default_rng(42)
_T_NP = _rng.standard_normal((A, SO, SI)).astype(np.float32) / np.sqrt(SI)
_MU = np.linspace(0.0, 2.0, R, dtype=np.float32)
_SIGMA = 0.5

_NC = 2
_NS = 16
_NW = _NC * _NS
_CHUNK = 128

_E1 = 81920
_E2 = E - _E1


def _mesh():
    return plsc.VectorSubcoreMesh(core_axis_name="c", subcore_axis_name="s",
                                  num_cores=_NC, num_subcores=_NS)


def _gather_body(per_w, nfull, tail,
                 x_hbm, src_hbm, out_hbm, idx_v, rows_v, rows_t, sem):
    wid = lax.axis_index("s") * _NC + lax.axis_index("c")
    base = wid * per_w
    pltpu.sync_copy(src_hbm.at[pl.ds(base, per_w)], idx_v)

    def body(i, _):
        off = i * _CHUNK
        idx = idx_v.at[pl.ds(off, _CHUNK)]
        pltpu.async_copy(x_hbm.at[idx], rows_v, sem).wait()
        pltpu.sync_copy(rows_v, out_hbm.at[pl.ds(base + off, _CHUNK)])
        return 0

    lax.fori_loop(0, nfull, body, 0)
    if tail:
        toff = nfull * _CHUNK
        tidx = idx_v.at[pl.ds(toff, tail)]
        pltpu.async_copy(x_hbm.at[tidx], rows_t, sem).wait()
        pltpu.sync_copy(rows_t, out_hbm.at[pl.ds(base + toff, tail)])


def _gather_call(x, src, e_sub):
    per_w = e_sub // _NW
    nfull, tail = divmod(per_w, _CHUNK)
    f = functools.partial(
        pl.kernel,
        out_type=jax.ShapeDtypeStruct((e_sub, D_IN), jnp.float32),
        mesh=_mesh(),
        scratch_types=[
            pltpu.VMEM((per_w,), jnp.int32),
            pltpu.VMEM((_CHUNK, D_IN), jnp.float32),
            pltpu.VMEM((max(tail, 8), D_IN), jnp.float32),
            pltpu.SemaphoreType.DMA,
        ],
    )(functools.partial(_gather_body, per_w, nfull, tail))
    return f(x, src)


_P = 640


def _compute_body(xj_ref, pts_ref, w_ref, out_ref):
    xj = xj_ref[...]
    pts = pts_ref[...]
    px = pts[:, 0:1]
    py = pts[:, 1:2]
    pz = pts[:, 2:3]
    r = jnp.sqrt(px * px + py * py + pz * pz)
    inv = 1.0 / (r + 1e-8)
    nx = px * inv
    ny = py * inv
    nz = pz * inv
    ms = [
        None,
        nx.astype(jnp.bfloat16),
        ny.astype(jnp.bfloat16),
        nz.astype(jnp.bfloat16),
        (nx * ny).astype(jnp.bfloat16),
        (nz * nz - jnp.float32(1.0 / 3.0)).astype(jnp.bfloat16),
    ]
    gs = [jnp.exp(-((r - jnp.float32(mu)) ** 2) * jnp.float32(1.0 / (2.0 * _SIGMA**2)))
          for mu in _MU]

    xjb = xj.astype(jnp.bfloat16)
    t3 = jnp.concatenate(
        [xjb] + [xjb * ms[a] for a in range(1, A)], axis=1)
    h = jnp.dot(t3, w_ref[...], preferred_element_type=jnp.float32)
    acc = gs[0] * h[:, 0:D_OUT]
    for rr in range(1, R):
        acc = acc + gs[rr] * h[:, rr * D_OUT:(rr + 1) * D_OUT]
    out_ref[...] = acc


def _compute_msg(xj, points, wbig):
    e_sub = xj.shape[0]
    return pl.pallas_call(
        _compute_body,
        grid=(e_sub // _P,),
        in_specs=[
            pl.BlockSpec((_P, D_IN), lambda i: (i, 0)),
            pl.BlockSpec((_P, 3), lambda i: (i, 0)),
            pl.BlockSpec((A * D_IN, R * D_OUT), lambda i: (0, 0)),
        ],
        out_specs=pl.BlockSpec((_P, D_OUT), lambda i: (i, 0)),
        out_shape=jax.ShapeDtypeStruct((e_sub, D_OUT), jnp.float32),
    )(xj, points, wbig)


_ROWS_PER_SUB = 624
_ROWS_TAIL_OFF = _ROWS_PER_SUB * _NS
_ROWS_TAIL = N - _ROWS_TAIL_OFF


def _scatter_body(per_w, nfull, tail,
                  msg_hbm, dst_hbm, zeros_hbm, out_hbm,
                  idx2_v, idxt_v, rows_v, rows_t, part_sh):
    cid = lax.axis_index("c")
    sid = lax.axis_index("s")
    wid = sid * _NC + cid
    pltpu.sync_copy(zeros_hbm.at[pl.ds(sid * _ROWS_PER_SUB, _ROWS_PER_SUB)],
                    part_sh.at[pl.ds(sid * _ROWS_PER_SUB, _ROWS_PER_SUB)])

    @pl.when(sid == _NS - 1)
    def _zero_tail():
        pltpu.sync_copy(zeros_hbm.at[pl.ds(_ROWS_TAIL_OFF, _ROWS_TAIL)],
                        part_sh.at[pl.ds(_ROWS_TAIL_OFF, _ROWS_TAIL)])

    plsc.subcore_barrier()

    base = wid * per_w

    def body(i, _):
        off = base + i * _CHUNK
        pltpu.sync_copy(dst_hbm.at[pl.ds(off, _CHUNK)], idx2_v.at[0])
        pltpu.sync_copy(msg_hbm.at[pl.ds(off, _CHUNK)], rows_v)
        pltpu.sync_copy(rows_v, part_sh.at[idx2_v.at[0]], add=True)
        return 0

    lax.fori_loop(0, nfull, body, 0)
    if tail:
        toff = base + nfull * _CHUNK
        pltpu.sync_copy(dst_hbm.at[pl.ds(toff, tail)], idxt_v)
        pltpu.sync_copy(msg_hbm.at[pl.ds(toff, tail)], rows_t)
        pltpu.sync_copy(rows_t, part_sh.at[idxt_v], add=True)

    plsc.subcore_barrier()
    row = sid * _ROWS_PER_SUB
    pltpu.sync_copy(part_sh.at[pl.ds(row, _ROWS_PER_SUB)],
                    out_hbm.at[pl.ds(cid * N + row, _ROWS_PER_SUB)])

    @pl.when(sid == _NS - 1)
    def _write_tail():
        pltpu.sync_copy(part_sh.at[pl.ds(_ROWS_TAIL_OFF, _ROWS_TAIL)],
                        out_hbm.at[pl.ds(cid * N + _ROWS_TAIL_OFF, _ROWS_TAIL)])


def _scatter_call(msg, dst, zeros):
    e_sub = msg.shape[0]
    per_w = e_sub // _NW
    nfull, tail = divmod(per_w, _CHUNK)
    f = functools.partial(
        pl.kernel,
        out_type=jax.ShapeDtypeStruct((2 * N, D_OUT), jnp.float32),
        mesh=_mesh(),
        scratch_types=[
            pltpu.VMEM((1, _CHUNK), jnp.int32),
            pltpu.VMEM((max(tail, 8),), jnp.int32),
            pltpu.VMEM((_CHUNK, D_OUT), jnp.float32),
            pltpu.VMEM((max(tail, 8), D_OUT), jnp.float32),
            pltpu.VMEM_SHARED((N, D_OUT), jnp.float32),
        ],
    )(functools.partial(_scatter_body, per_w, nfull, tail))
    return f(msg, dst, zeros)


_MB = 1000


def _merge_body(a0_ref, a1_ref, b0_ref, b1_ref, out_ref):
    out_ref[...] = (a0_ref[...] + a1_ref[...]) + (b0_ref[...] + b1_ref[...])


def _merge(pa, pb):
    nb = N // _MB
    return pl.pallas_call(
        _merge_body,
        grid=(nb,),
        in_specs=[
            pl.BlockSpec((_MB, D_OUT), lambda i: (i, 0)),
            pl.BlockSpec((_MB, D_OUT), lambda i: (i + nb, 0)),
            pl.BlockSpec((_MB, D_OUT), lambda i: (i, 0)),
            pl.BlockSpec((_MB, D_OUT), lambda i: (i + nb, 0)),
        ],
        out_specs=pl.BlockSpec((_MB, D_OUT), lambda i: (i, 0)),
        out_shape=jax.ShapeDtypeStruct((N, D_OUT), jnp.float32),
    )(pa, pa, pb, pb)


def _build_wbig(weights):
    c = weights.reshape(CO, CI, R, A)
    c_t = jnp.transpose(c, (3, 1, 2, 0))
    t_t = jnp.transpose(jnp.asarray(_T_NP), (0, 2, 1))
    w = (c_t[:, :, None, :, :, None] * t_t[:, None, :, None, None, :])
    return w.reshape(A * D_IN, R * D_OUT).astype(jnp.bfloat16)


def kernel(weights, x, points, edge_index):
    src = edge_index[0].astype(jnp.int32)
    dst = edge_index[1].astype(jnp.int32)
    wbig = _build_wbig(weights)
    zeros = jnp.zeros((N, D_OUT), dtype=jnp.float32)

    xj1 = _gather_call(x, src[:_E1], _E1)
    xj2 = _gather_call(x, src[_E1:], _E2)
    msg1 = _compute_msg(xj1, points[:_E1], wbig)
    msg2 = _compute_msg(xj2, points[_E1:], wbig)
    pa = _scatter_call(msg1, dst[:_E1], zeros)
    pb = _scatter_call(msg2, dst[_E1:], zeros)
    return _merge(pa, pb)


# speedup vs baseline: 21.1251x; 1.0807x over previous
import functools

import jax
import jax.numpy as jnp
import numpy as np
from jax import lax
from jax.experimental import pallas as pl
from jax.experimental.pallas import tpu as pltpu
from jax.experimental.pallas import tpu_sc as plsc

N = 10000
E = 160000
CI = 16
CO = 16
SI = 8
SO = 8
R = 8
A = 6
K = R * A
D_IN = CI * SI
D_OUT = CO * SO

_rng = np.random.default_rng(42)
_T_NP = _rng.standard_normal((A, SO, SI)).astype(np.float32) / np.sqrt(SI)
_MU = np.linspace(0.0, 2.0, R, dtype=np.float32)
_SIGMA = 0.5

_NC = 2
_NS = 16
_NW = _NC * _NS
_CHUNK = 128

_E1 = 81920
_E2 = E - _E1


def _mesh():
    return plsc.VectorSubcoreMesh(core_axis_name="c", subcore_axis_name="s",
                                  num_cores=_NC, num_subcores=_NS)


def _gather_body(per_w, nfull, tail,
                 x_hbm, src_hbm, out_hbm, idx_v, rows_v, rows_t, sem):
    wid = lax.axis_index("s") * _NC + lax.axis_index("c")
    base = wid * per_w
    pltpu.sync_copy(src_hbm.at[pl.ds(base, per_w)], idx_v)

    def body(i, _):
        off = i * _CHUNK
        idx = idx_v.at[pl.ds(off, _CHUNK)]
        pltpu.async_copy(x_hbm.at[idx], rows_v, sem).wait()
        pltpu.sync_copy(rows_v, out_hbm.at[pl.ds(base + off, _CHUNK)])
        return 0

    lax.fori_loop(0, nfull, body, 0)
    if tail:
        toff = nfull * _CHUNK
        tidx = idx_v.at[pl.ds(toff, tail)]
        pltpu.async_copy(x_hbm.at[tidx], rows_t, sem).wait()
        pltpu.sync_copy(rows_t, out_hbm.at[pl.ds(base + toff, tail)])


def _gather_call(x, src, e_sub):
    per_w = e_sub // _NW
    nfull, tail = divmod(per_w, _CHUNK)
    f = functools.partial(
        pl.kernel,
        out_type=jax.ShapeDtypeStruct((e_sub, D_IN), jnp.float32),
        mesh=_mesh(),
        scratch_types=[
            pltpu.VMEM((per_w,), jnp.int32),
            pltpu.VMEM((_CHUNK, D_IN), jnp.float32),
            pltpu.VMEM((max(tail, 8), D_IN), jnp.float32),
            pltpu.SemaphoreType.DMA,
        ],
    )(functools.partial(_gather_body, per_w, nfull, tail))
    return f(x, src)


_P = 1280


def _compute_body(xj_ref, pts_ref, w_ref, out_ref):
    xj = xj_ref[...]
    pts = pts_ref[...]
    px = pts[:, 0:1]
    py = pts[:, 1:2]
    pz = pts[:, 2:3]
    r = jnp.sqrt(px * px + py * py + pz * pz)
    inv = 1.0 / (r + 1e-8)
    nx = px * inv
    ny = py * inv
    nz = pz * inv
    ms = [
        None,
        nx.astype(jnp.bfloat16),
        ny.astype(jnp.bfloat16),
        nz.astype(jnp.bfloat16),
        (nx * ny).astype(jnp.bfloat16),
        (nz * nz - jnp.float32(1.0 / 3.0)).astype(jnp.bfloat16),
    ]
    gs = [jnp.exp(-((r - jnp.float32(mu)) ** 2) * jnp.float32(1.0 / (2.0 * _SIGMA**2)))
          for mu in _MU]

    xjb = xj.astype(jnp.bfloat16)
    t3 = jnp.concatenate(
        [xjb] + [xjb * ms[a] for a in range(1, A)], axis=1)
    h = jnp.dot(t3, w_ref[...], preferred_element_type=jnp.float32)
    acc = gs[0] * h[:, 0:D_OUT]
    for rr in range(1, R):
        acc = acc + gs[rr] * h[:, rr * D_OUT:(rr + 1) * D_OUT]
    out_ref[...] = acc


def _compute_msg(xj, points, wbig):
    e_sub = xj.shape[0]
    return pl.pallas_call(
        _compute_body,
        grid=(e_sub // _P,),
        in_specs=[
            pl.BlockSpec((_P, D_IN), lambda i: (i, 0)),
            pl.BlockSpec((_P, 3), lambda i: (i, 0)),
            pl.BlockSpec((A * D_IN, R * D_OUT), lambda i: (0, 0)),
        ],
        out_specs=pl.BlockSpec((_P, D_OUT), lambda i: (i, 0)),
        out_shape=jax.ShapeDtypeStruct((e_sub, D_OUT), jnp.float32),
    )(xj, points, wbig)


_ROWS_PER_SUB = 624
_ROWS_TAIL_OFF = _ROWS_PER_SUB * _NS
_ROWS_TAIL = N - _ROWS_TAIL_OFF


def _scatter_body(per_w, nfull, tail,
                  msg_hbm, dst_hbm, zeros_hbm, out_hbm,
                  idx2_v, idxt_v, rows_v, rows_t, part_sh):
    cid = lax.axis_index("c")
    sid = lax.axis_index("s")
    wid = sid * _NC + cid
    pltpu.sync_copy(zeros_hbm.at[pl.ds(sid * _ROWS_PER_SUB, _ROWS_PER_SUB)],
                    part_sh.at[pl.ds(sid * _ROWS_PER_SUB, _ROWS_PER_SUB)])

    @pl.when(sid == _NS - 1)
    def _zero_tail():
        pltpu.sync_copy(zeros_hbm.at[pl.ds(_ROWS_TAIL_OFF, _ROWS_TAIL)],
                        part_sh.at[pl.ds(_ROWS_TAIL_OFF, _ROWS_TAIL)])

    plsc.subcore_barrier()

    base = wid * per_w

    def body(i, _):
        off = base + i * _CHUNK
        pltpu.sync_copy(dst_hbm.at[pl.ds(off, _CHUNK)], idx2_v.at[0])
        pltpu.sync_copy(msg_hbm.at[pl.ds(off, _CHUNK)], rows_v)
        pltpu.sync_copy(rows_v, part_sh.at[idx2_v.at[0]], add=True)
        return 0

    lax.fori_loop(0, nfull, body, 0)
    if tail:
        toff = base + nfull * _CHUNK
        pltpu.sync_copy(dst_hbm.at[pl.ds(toff, tail)], idxt_v)
        pltpu.sync_copy(msg_hbm.at[pl.ds(toff, tail)], rows_t)
        pltpu.sync_copy(rows_t, part_sh.at[idxt_v], add=True)

    plsc.subcore_barrier()
    row = sid * _ROWS_PER_SUB
    pltpu.sync_copy(part_sh.at[pl.ds(row, _ROWS_PER_SUB)],
                    out_hbm.at[pl.ds(cid * N + row, _ROWS_PER_SUB)])

    @pl.when(sid == _NS - 1)
    def _write_tail():
        pltpu.sync_copy(part_sh.at[pl.ds(_ROWS_TAIL_OFF, _ROWS_TAIL)],
                        out_hbm.at[pl.ds(cid * N + _ROWS_TAIL_OFF, _ROWS_TAIL)])


def _scatter_call(msg, dst, zeros):
    e_sub = msg.shape[0]
    per_w = e_sub // _NW
    nfull, tail = divmod(per_w, _CHUNK)
    f = functools.partial(
        pl.kernel,
        out_type=jax.ShapeDtypeStruct((2 * N, D_OUT), jnp.float32),
        mesh=_mesh(),
        scratch_types=[
            pltpu.VMEM((1, _CHUNK), jnp.int32),
            pltpu.VMEM((max(tail, 8),), jnp.int32),
            pltpu.VMEM((_CHUNK, D_OUT), jnp.float32),
            pltpu.VMEM((max(tail, 8), D_OUT), jnp.float32),
            pltpu.VMEM_SHARED((N, D_OUT), jnp.float32),
        ],
    )(functools.partial(_scatter_body, per_w, nfull, tail))
    return f(msg, dst, zeros)


_MB = 1000


def _merge_body(a0_ref, a1_ref, b0_ref, b1_ref, out_ref):
    out_ref[...] = (a0_ref[...] + a1_ref[...]) + (b0_ref[...] + b1_ref[...])


def _merge(pa, pb):
    nb = N // _MB
    return pl.pallas_call(
        _merge_body,
        grid=(nb,),
        in_specs=[
            pl.BlockSpec((_MB, D_OUT), lambda i: (i, 0)),
            pl.BlockSpec((_MB, D_OUT), lambda i: (i + nb, 0)),
            pl.BlockSpec((_MB, D_OUT), lambda i: (i, 0)),
            pl.BlockSpec((_MB, D_OUT), lambda i: (i + nb, 0)),
        ],
        out_specs=pl.BlockSpec((_MB, D_OUT), lambda i: (i, 0)),
        out_shape=jax.ShapeDtypeStruct((N, D_OUT), jnp.float32),
    )(pa, pa, pb, pb)


def _build_wbig(weights):
    c = weights.reshape(CO, CI, R, A)
    c_t = jnp.transpose(c, (3, 1, 2, 0))
    t_t = jnp.transpose(jnp.asarray(_T_NP), (0, 2, 1))
    w = (c_t[:, :, None, :, :, None] * t_t[:, None, :, None, None, :])
    return w.reshape(A * D_IN, R * D_OUT).astype(jnp.bfloat16)


def kernel(weights, x, points, edge_index):
    src = edge_index[0].astype(jnp.int32)
    dst = edge_index[1].astype(jnp.int32)
    wbig = _build_wbig(weights)
    zeros = jnp.zeros((N, D_OUT), dtype=jnp.float32)

    xj1 = _gather_call(x, src[:_E1], _E1)
    xj2 = _gather_call(x, src[_E1:], _E2)
    msg1 = _compute_msg(xj1, points[:_E1], wbig)
    msg2 = _compute_msg(xj2, points[_E1:], wbig)
    pa = _scatter_call(msg1, dst[:_E1], zeros)
    pb = _scatter_call(msg2, dst[_E1:], zeros)
    return _merge(pa, pb)
